# baseline (speedup 1.0000x reference)
import functools

import jax
import jax.numpy as jnp
from jax import lax
from jax.experimental import pallas as pl
from jax.experimental.pallas import tpu as pltpu
from jax.experimental.pallas import tpu_sc as plsc

B = 4096
NEG = 49
NCAND = NEG + 1
RANK = 128
NC, NS, L = 2, 16, 16
NW = NC * NS
RPW = B // NW
RB = 2
NBLK = RPW // RB
NHALF = NBLK // 2
TROWS = RB * NCAND
K = RANK // L
NG = NCAND // L
NTAIL = NCAND - NG * L


@functools.lru_cache(maxsize=None)
def _build():
    mesh = plsc.VectorSubcoreMesh(
        core_axis_name="c", subcore_axis_name="s",
        num_cores=NC, num_subcores=NS)

    @functools.partial(
        pl.kernel,
        out_type=(
            jax.ShapeDtypeStruct((B * NCAND,), jnp.float32),
            jax.ShapeDtypeStruct((B, RANK), jnp.float32),
            jax.ShapeDtypeStruct((B, RANK), jnp.float32),
            jax.ShapeDtypeStruct((B, RANK), jnp.float32),
            jax.ShapeDtypeStruct((B, RANK), jnp.float32),
        ),
        mesh=mesh,
        compiler_params=pltpu.CompilerParams(needs_layout_passes=False),
        scratch_types=[
            pltpu.VMEM((RPW,), jnp.int32),
            pltpu.VMEM((RPW,), jnp.int32),
            pltpu.VMEM((RPW,), jnp.int32),
            pltpu.VMEM((RPW,), jnp.int32),
            pltpu.VMEM((NBLK, TROWS), jnp.int32),
            pltpu.VMEM((RPW, RANK), jnp.float32),
            pltpu.VMEM((RPW, RANK), jnp.float32),
            pltpu.VMEM((RPW, RANK), jnp.float32),
            pltpu.VMEM((RPW, RANK), jnp.float32),
            pltpu.VMEM((TROWS, RANK), jnp.float32),
            pltpu.VMEM((TROWS, RANK), jnp.float32),
            pltpu.VMEM((RPW * NCAND + L,), jnp.float32),
            pltpu.SemaphoreType.DMA,
            pltpu.SemaphoreType.DMA,
            pltpu.SemaphoreType.DMA,
            pltpu.SemaphoreType.DMA,
        ],
    )
    def hyte(ent_hbm, rel_hbm, time_hbm, sidx_hbm, ridx_hbm, tidx_hbm,
             pidx_hbm, oidx_hbm,
             scores_hbm, reg0_hbm, reg1_hbm, reg2_hbm, te_hbm,
             sidx_v, ridx_v, tidx_v, pidx_v, oidx_v,
             h_v, r_v, tm_v, po_v, tl0_v, tl1_v, sc_v,
             sem0, sem_a, sem_b, sem_w):
        wid = lax.axis_index("s") * NC + lax.axis_index("c")
        base = wid * RPW
        lanes = lax.iota(jnp.int32, L)
        bfly = [jnp.bitwise_xor(lanes, dd) for dd in (1, 2, 4, 8)]

        def allsum(v):
            for perm in bfly:
                v = v + jnp.take_along_axis(
                    v, perm, axis=0, mode="promise_in_bounds")
            return v

        pltpu.sync_copy(sidx_hbm.at[pl.ds(base, RPW)], sidx_v)
        pltpu.sync_copy(ridx_hbm.at[pl.ds(base, RPW)], ridx_v)
        pltpu.sync_copy(tidx_hbm.at[pl.ds(base, RPW)], tidx_v)
        pltpu.sync_copy(pidx_hbm.at[pl.ds(base, RPW)], pidx_v)
        pltpu.sync_copy(oidx_hbm.at[wid], oidx_v)

        c0 = pltpu.async_copy(ent_hbm.at[sidx_v], h_v, sem0)
        c1 = pltpu.async_copy(rel_hbm.at[ridx_v], r_v, sem0)
        c2 = pltpu.async_copy(time_hbm.at[tidx_v], tm_v, sem0)
        c3 = pltpu.async_copy(ent_hbm.at[pidx_v], po_v, sem0)
        c0.wait(); c1.wait(); c2.wait(); c3.wait()

        w0 = pltpu.async_copy(h_v, reg0_hbm.at[pl.ds(base, RPW)], sem_w)
        w1 = pltpu.async_copy(r_v, reg1_hbm.at[pl.ds(base, RPW)], sem_w)
        w2 = pltpu.async_copy(po_v, reg2_hbm.at[pl.ds(base, RPW)], sem_w)
        w3 = pltpu.async_copy(tm_v, te_hbm.at[pl.ds(base, RPW)], sem_w)

        pltpu.async_copy(ent_hbm.at[oidx_v.at[0]], tl0_v, sem_a)
        pltpu.async_copy(ent_hbm.at[oidx_v.at[1]], tl1_v, sem_b)

        def compute_block(tl_v, b):
            for i in range(RB):
                row = b * RB + i
                tmk = [tm_v[row, pl.ds(L * k, L)] for k in range(K)]
                ak = [h_v[row, pl.ds(L * k, L)] + r_v[row, pl.ds(L * k, L)]
                      for k in range(K)]
                dv = ak[0] * tmk[0]
                for k in range(1, K):
                    dv = dv + ak[k] * tmk[k]
                d = allsum(dv)
                pk = [ak[k] - d * tmk[k] for k in range(K)]

                def pair(trow):
                    g = [tl_v[trow, pl.ds(L * k, L)] for k in range(K)]
                    pg = [pk[k] - g[k] for k in range(K)]
                    cv = g[0] * tmk[0]
                    for k in range(1, K):
                        cv = cv + g[k] * tmk[k]
                    c = allsum(cv)
                    sv = jnp.abs(pg[0] + c * tmk[0])
                    for k in range(1, K):
                        sv = sv + jnp.abs(pg[k] + c * tmk[k])
                    return allsum(sv)

                def group_body(gi, vec):
                    half = (gi & 1) * 8
                    for u in range(8):
                        s = pair(i * NCAND + gi * 8 + u)
                        vec = jnp.where(lanes == half + u, s, vec)

                    @pl.when((gi & 1) == 1)
                    def _store():
                        sc_v[pl.ds(row * NCAND + (gi - 1) * 8, L)] = vec
                    return vec

                lax.fori_loop(0, NG * 2, group_body,
                              jnp.zeros((L,), jnp.float32))

                st = [pair(i * NCAND + NG * L + u) for u in range(NTAIL)]
                vec = jnp.where(lanes == 0, st[0], 0.0)
                for u in range(1, NTAIL):
                    vec = jnp.where(lanes == u, st[u], vec)
                sc_v[pl.ds(row * NCAND + NG * L, L)] = vec

        def loop_body(m, carry):
            b0 = 2 * m
            plsc.subcore_barrier()
            pltpu.make_async_copy(ent_hbm.at[oidx_v.at[b0]], tl0_v,
                                  sem_a).wait()
            compute_block(tl0_v, b0)

            @pl.when(m < NHALF - 1)
            def _pf0():
                pltpu.async_copy(ent_hbm.at[oidx_v.at[b0 + 2]], tl0_v, sem_a)

            pltpu.make_async_copy(ent_hbm.at[oidx_v.at[b0 + 1]], tl1_v,
                                  sem_b).wait()
            plsc.subcore_barrier()
            compute_block(tl1_v, b0 + 1)

            @pl.when(m < NHALF - 1)
            def _pf1():
                pltpu.async_copy(ent_hbm.at[oidx_v.at[b0 + 3]], tl1_v, sem_b)

            return carry

        lax.fori_loop(0, NHALF, loop_body, 0)

        pltpu.sync_copy(sc_v.at[pl.ds(0, RPW * NCAND)],
                        scores_hbm.at[pl.ds(base * NCAND, RPW * NCAND)])
        w0.wait(); w1.wait(); w2.wait(); w3.wait()

    return hyte


def kernel(x, weight, ent_w, rel_w, time_w):
    x32 = x.astype(jnp.int32)
    neg = jax.random.randint(
        jax.random.key(42), (x.shape[0], NEG), 0, ent_w.shape[0]
    ).astype(jnp.int32)
    o_all = jnp.concatenate([x32[:, 2:3], neg], axis=1)
    o3 = o_all.reshape(NW, NBLK, TROWS)
    fn = _build()
    scores_flat, reg0, reg1, reg2, te = fn(
        ent_w, rel_w, time_w,
        x32[:, 0], x32[:, 1], x32[:, 3], x32[:, 2], o3)
    return (scores_flat.reshape(B, NCAND), reg0, reg1, reg2, te)


# speedup vs baseline: 1.0159x; 1.0159x over previous
import functools

import jax
import jax.numpy as jnp
from jax import lax
from jax.experimental import pallas as pl
from jax.experimental.pallas import tpu as pltpu
from jax.experimental.pallas import tpu_sc as plsc

B = 4096
NEG = 49
NCAND = NEG + 1
RANK = 128
NC, NS, L = 2, 16, 16
NW = NC * NS
RPW = B // NW
RB = 2
NBLK = RPW // RB
NHALF = NBLK // 2
TROWS = RB * NCAND
K = RANK // L
NG = NCAND // L
NTAIL = NCAND - NG * L


@functools.lru_cache(maxsize=None)
def _build():
    mesh = plsc.VectorSubcoreMesh(
        core_axis_name="c", subcore_axis_name="s",
        num_cores=NC, num_subcores=NS)

    @functools.partial(
        pl.kernel,
        out_type=(
            jax.ShapeDtypeStruct((B * NCAND,), jnp.float32),
            jax.ShapeDtypeStruct((B, RANK), jnp.float32),
            jax.ShapeDtypeStruct((B, RANK), jnp.float32),
            jax.ShapeDtypeStruct((B, RANK), jnp.float32),
            jax.ShapeDtypeStruct((B, RANK), jnp.float32),
        ),
        mesh=mesh,
        compiler_params=pltpu.CompilerParams(needs_layout_passes=False),
        scratch_types=[
            pltpu.VMEM((4, RPW), jnp.int32),
            pltpu.VMEM((NBLK, TROWS), jnp.int32),
            pltpu.VMEM((RPW, RANK), jnp.float32),
            pltpu.VMEM((RPW, RANK), jnp.float32),
            pltpu.VMEM((RPW, RANK), jnp.float32),
            pltpu.VMEM((RPW, RANK), jnp.float32),
            pltpu.VMEM((TROWS, RANK), jnp.float32),
            pltpu.VMEM((TROWS, RANK), jnp.float32),
            pltpu.VMEM((RPW * NCAND + L,), jnp.float32),
            pltpu.SemaphoreType.DMA,
            pltpu.SemaphoreType.DMA,
            pltpu.SemaphoreType.DMA,
            pltpu.SemaphoreType.DMA,
        ],
    )
    def hyte(ent_hbm, rel_hbm, time_hbm, idx4_hbm, oidx_hbm,
             scores_hbm, reg0_hbm, reg1_hbm, reg2_hbm, te_hbm,
             idx4_v, oidx_v,
             h_v, r_v, tm_v, po_v, tl0_v, tl1_v, sc_v,
             sem0, sem_a, sem_b, sem_w):
        wid = lax.axis_index("s") * NC + lax.axis_index("c")
        base = wid * RPW
        lanes = lax.iota(jnp.int32, L)
        bfly = [jnp.bitwise_xor(lanes, dd) for dd in (1, 2, 4, 8)]

        def allsum(v):
            for perm in bfly:
                v = v + jnp.take_along_axis(
                    v, perm, axis=0, mode="promise_in_bounds")
            return v

        pltpu.sync_copy(idx4_hbm.at[wid], idx4_v)
        co = pltpu.async_copy(oidx_hbm.at[wid], oidx_v, sem0)

        c0 = pltpu.async_copy(ent_hbm.at[idx4_v.at[0]], h_v, sem0)
        c1 = pltpu.async_copy(rel_hbm.at[idx4_v.at[1]], r_v, sem0)
        c2 = pltpu.async_copy(time_hbm.at[idx4_v.at[2]], tm_v, sem0)
        c3 = pltpu.async_copy(ent_hbm.at[idx4_v.at[3]], po_v, sem0)
        co.wait(); c0.wait(); c1.wait(); c2.wait(); c3.wait()

        w0 = pltpu.async_copy(h_v, reg0_hbm.at[pl.ds(base, RPW)], sem_w)
        w1 = pltpu.async_copy(r_v, reg1_hbm.at[pl.ds(base, RPW)], sem_w)
        w2 = pltpu.async_copy(po_v, reg2_hbm.at[pl.ds(base, RPW)], sem_w)
        w3 = pltpu.async_copy(tm_v, te_hbm.at[pl.ds(base, RPW)], sem_w)

        pltpu.async_copy(ent_hbm.at[oidx_v.at[0]], tl0_v, sem_a)
        pltpu.async_copy(ent_hbm.at[oidx_v.at[1]], tl1_v, sem_b)

        def compute_block(tl_v, b):
            for i in range(RB):
                row = b * RB + i
                tmk = [tm_v[row, pl.ds(L * k, L)] for k in range(K)]
                ak = [h_v[row, pl.ds(L * k, L)] + r_v[row, pl.ds(L * k, L)]
                      for k in range(K)]
                dv = ak[0] * tmk[0]
                for k in range(1, K):
                    dv = dv + ak[k] * tmk[k]
                d = allsum(dv)
                pk = [ak[k] - d * tmk[k] for k in range(K)]

                def pair(trow):
                    g = [tl_v[trow, pl.ds(L * k, L)] for k in range(K)]
                    pg = [pk[k] - g[k] for k in range(K)]
                    cv = g[0] * tmk[0]
                    for k in range(1, K):
                        cv = cv + g[k] * tmk[k]
                    c = allsum(cv)
                    sv = jnp.abs(pg[0] + c * tmk[0])
                    for k in range(1, K):
                        sv = sv + jnp.abs(pg[k] + c * tmk[k])
                    return allsum(sv)

                @plsc.parallel_loop(0, NG)
                def group_body(gi):
                    g0 = gi * L
                    s = [pair(i * NCAND + g0 + u) for u in range(L)]
                    vec = jnp.where(lanes == 0, s[0], 0.0)
                    for u in range(1, L):
                        vec = jnp.where(lanes == u, s[u], vec)
                    sc_v[pl.ds(row * NCAND + g0, L)] = vec

                st = [pair(i * NCAND + NG * L + u) for u in range(NTAIL)]
                vec = jnp.where(lanes == 0, st[0], 0.0)
                for u in range(1, NTAIL):
                    vec = jnp.where(lanes == u, st[u], vec)
                sc_v[pl.ds(row * NCAND + NG * L, L)] = vec

        def loop_body(m, carry):
            b0 = 2 * m
            plsc.subcore_barrier()
            pltpu.make_async_copy(ent_hbm.at[oidx_v.at[b0]], tl0_v,
                                  sem_a).wait()
            compute_block(tl0_v, b0)

            @pl.when(m < NHALF - 1)
            def _pf0():
                pltpu.async_copy(ent_hbm.at[oidx_v.at[b0 + 2]], tl0_v, sem_a)

            pltpu.make_async_copy(ent_hbm.at[oidx_v.at[b0 + 1]], tl1_v,
                                  sem_b).wait()
            plsc.subcore_barrier()
            compute_block(tl1_v, b0 + 1)

            @pl.when(m < NHALF - 1)
            def _pf1():
                pltpu.async_copy(ent_hbm.at[oidx_v.at[b0 + 3]], tl1_v, sem_b)

            return carry

        lax.fori_loop(0, NHALF, loop_body, 0)

        pltpu.sync_copy(sc_v.at[pl.ds(0, RPW * NCAND)],
                        scores_hbm.at[pl.ds(base * NCAND, RPW * NCAND)])
        w0.wait(); w1.wait(); w2.wait(); w3.wait()

    return hyte


def kernel(x, weight, ent_w, rel_w, time_w):
    x32 = x.astype(jnp.int32)
    neg = jax.random.randint(
        jax.random.key(42), (x.shape[0], NEG), 0, ent_w.shape[0]
    ).astype(jnp.int32)
    o_all = jnp.concatenate([x32[:, 2:3], neg], axis=1)
    o3 = o_all.reshape(NW, NBLK, TROWS)
    idx4 = jnp.stack(
        [x32[:, 0], x32[:, 1], x32[:, 3], x32[:, 2]]
    ).reshape(4, NW, RPW).transpose(1, 0, 2)
    fn = _build()
    scores_flat, reg0, reg1, reg2, te = fn(
        ent_w, rel_w, time_w, idx4, o3)
    return (scores_flat.reshape(B, NCAND), reg0, reg1, reg2, te)
